# baseline (speedup 1.0000x reference)
import functools

import jax
import jax.numpy as jnp
from jax import lax
from jax.experimental import pallas as pl
from jax.experimental.pallas import tpu as pltpu
from jax.experimental.pallas import tpu_sc as plsc

_NC = 2
_NS = 16
_NW = _NC * _NS
_L = 16


def _build_sc_kernel(R, D, C, RPW, K):
    NCH = RPW // K
    num_pairs_rows = (R // C // 2) * C
    WPI = C // RPW
    WPI_SHIFT = WPI.bit_length() - 1
    assert WPI == 1 << WPI_SHIFT and C % RPW == 0
    mesh = plsc.VectorSubcoreMesh(core_axis_name="c", subcore_axis_name="s")

    @functools.partial(
        pl.kernel,
        mesh=mesh,
        compiler_params=pltpu.CompilerParams(needs_layout_passes=False),
        out_type=jax.ShapeDtypeStruct((R, D), jnp.float32),
        scratch_types=[
            pltpu.VMEM((RPW,), jnp.int32),
            pltpu.VMEM((RPW,), jnp.int32),
            pltpu.VMEM((num_pairs_rows,), jnp.int32),
            pltpu.VMEM((C,), jnp.int32),
            pltpu.VMEM((K, D), jnp.float32),
            pltpu.SemaphoreType.DMA,
        ],
    )
    def sc_k(y_hbm, mix_hbm, drop_hbm, out_hbm, idx_v, dropf_v, mix_v,
             drop_v, buf_v, sem):
        wid = lax.axis_index("s") * _NC + lax.axis_index("c")
        base = wid * RPW
        pltpu.sync_copy(mix_hbm, mix_v)
        pltpu.sync_copy(drop_hbm, drop_v)
        lanes = lax.iota(jnp.int32, _L)
        inst = wid >> WPI_SHIFT
        cbase = (wid & (WPI - 1)) * RPW
        for t in range(RPW // _L):
            c = cbase + t * _L + lanes
            r = inst * C + c
            partner = (inst ^ 1) * C + c
            mixf = plsc.load_gather(mix_v, [(inst >> 1) * C + c])
            dropf = plsc.load_gather(drop_v, [c])
            idx_v[pl.ds(t * _L, _L)] = jnp.where(mixf != 0, partner, r)
            dropf_v[pl.ds(t * _L, _L)] = dropf
        for g in range(NCH):
            pltpu.async_copy(
                y_hbm.at[idx_v.at[pl.ds(g * K, K)]], buf_v, sem).wait()
            df = dropf_v[pl.ds((g * K) // _L * _L, _L)]
            for j in range(K):
                lane = (g * K) % _L + j
                is_drop = jnp.sum(jnp.where(lanes == lane, df, 0))

                @pl.when(is_drop != 0)
                def _zero_row(j=j):
                    zv = jnp.zeros((_L,), jnp.float32)

                    def zbody(q, carry):
                        buf_v[j, pl.ds(q * _L, _L)] = zv
                        return carry

                    lax.fori_loop(0, D // _L, zbody, 0)

            pltpu.sync_copy(buf_v, out_hbm.at[pl.ds(base + g * K, K)])

    return sc_k


def kernel(y, mix_mask, drop_mask):
    bs, C, H, W = y.shape
    D = H * W
    R = bs * C
    num_pairs = bs // 2
    RPW = R // _NW
    K = 8
    y_flat = y.reshape(R, D)
    mix_i = mix_mask.astype(jnp.int32).reshape(num_pairs * C)
    drop_i = drop_mask.astype(jnp.int32)
    sc_k = _build_sc_kernel(R, D, C, RPW, K)
    out_flat = sc_k(y_flat, mix_i, drop_i)
    return out_flat.reshape(bs, C, H, W)


# speedup vs baseline: 1.0703x; 1.0703x over previous
import functools

import jax
import jax.numpy as jnp
from jax import lax
from jax.experimental import pallas as pl
from jax.experimental.pallas import tpu as pltpu
from jax.experimental.pallas import tpu_sc as plsc

_NC = 2
_NS = 16
_NW = _NC * _NS
_L = 16


def _build_sc_kernel(R, D, C, RPW, K):
    NCH = RPW // K
    NBUF = 3
    num_pairs_rows = (R // C // 2) * C
    WPI = C // RPW
    WPI_SHIFT = WPI.bit_length() - 1
    assert WPI == 1 << WPI_SHIFT and C % RPW == 0
    mesh = plsc.VectorSubcoreMesh(core_axis_name="c", subcore_axis_name="s")

    @functools.partial(
        pl.kernel,
        mesh=mesh,
        compiler_params=pltpu.CompilerParams(needs_layout_passes=False),
        out_type=jax.ShapeDtypeStruct((R, D), jnp.float32),
        scratch_types=[
            pltpu.VMEM((RPW,), jnp.int32),
            pltpu.VMEM((RPW,), jnp.int32),
            pltpu.VMEM((num_pairs_rows,), jnp.int32),
            pltpu.VMEM((C,), jnp.int32),
            pltpu.VMEM((NBUF * K, D), jnp.float32),
            pltpu.SemaphoreType.DMA((NBUF,)),
            pltpu.SemaphoreType.DMA((NBUF,)),
        ],
    )
    def sc_k(y_hbm, mix_hbm, drop_hbm, out_hbm, idx_v, dropf_v, mix_v,
             drop_v, buf_v, gsem, wsem):
        wid = lax.axis_index("s") * _NC + lax.axis_index("c")
        base = wid * RPW
        pltpu.sync_copy(mix_hbm, mix_v)
        pltpu.sync_copy(drop_hbm, drop_v)
        lanes = lax.iota(jnp.int32, _L)
        inst = wid >> WPI_SHIFT
        cbase = (wid & (WPI - 1)) * RPW
        for t in range(RPW // _L):
            c = cbase + t * _L + lanes
            r = inst * C + c
            partner = (inst ^ 1) * C + c
            mixf = plsc.load_gather(mix_v, [(inst >> 1) * C + c])
            dropf = plsc.load_gather(drop_v, [c])
            idx_v[pl.ds(t * _L, _L)] = jnp.where(mixf != 0, partner, r)
            dropf_v[pl.ds(t * _L, _L)] = dropf
        def start_gather(g):
            b = g % NBUF
            pltpu.async_copy(
                y_hbm.at[idx_v.at[pl.ds(g * K, K)]],
                buf_v.at[pl.ds(b * K, K)], gsem.at[b])

        def start_write(g):
            b = g % NBUF
            pltpu.async_copy(
                buf_v.at[pl.ds(b * K, K)],
                out_hbm.at[pl.ds(base + g * K, K)], wsem.at[b])

        def wait_gather(g):
            b = g % NBUF
            pltpu.make_async_copy(
                y_hbm.at[idx_v.at[pl.ds(g * K, K)]],
                buf_v.at[pl.ds(b * K, K)], gsem.at[b]).wait()

        def wait_write(g):
            b = g % NBUF
            pltpu.make_async_copy(
                buf_v.at[pl.ds(b * K, K)],
                out_hbm.at[pl.ds(base + g * K, K)], wsem.at[b]).wait()

        for g in range(min(NBUF, NCH)):
            start_gather(g)
        for g in range(NCH):
            b = g % NBUF
            if g >= 1 and g - 1 + NBUF < NCH:
                wait_write(g - 1)
                start_gather(g - 1 + NBUF)
            wait_gather(g)
            df = dropf_v[pl.ds((g * K) // _L * _L, _L)]
            for j in range(K):
                lane = (g * K) % _L + j
                is_drop = jnp.sum(jnp.where(lanes == lane, df, 0))

                @pl.when(is_drop != 0)
                def _zero_row(b=b, j=j):
                    zv = jnp.zeros((_L,), jnp.float32)

                    def zbody(q, carry):
                        buf_v[b * K + j, pl.ds(q * _L, _L)] = zv
                        return carry

                    lax.fori_loop(0, D // _L, zbody, 0)

            start_write(g)
        for g in range(max(NCH - NBUF, 0), NCH):
            wait_write(g)

    return sc_k


def kernel(y, mix_mask, drop_mask):
    bs, C, H, W = y.shape
    D = H * W
    R = bs * C
    num_pairs = bs // 2
    RPW = R // _NW
    K = 8
    y_flat = y.reshape(R, D)
    mix_i = mix_mask.astype(jnp.int32).reshape(num_pairs * C)
    drop_i = drop_mask.astype(jnp.int32)
    sc_k = _build_sc_kernel(R, D, C, RPW, K)
    out_flat = sc_k(y_flat, mix_i, drop_i)
    return out_flat.reshape(bs, C, H, W)


# speedup vs baseline: 1.3098x; 1.2238x over previous
import functools

import jax
import jax.numpy as jnp
from jax import lax
from jax.experimental import pallas as pl
from jax.experimental.pallas import tpu as pltpu
from jax.experimental.pallas import tpu_sc as plsc

_NC = 2
_NS = 16
_NW = _NC * _NS
_L = 16


def _build_sc_kernel(bs, C, H, W, RPW, K):
    NCH = RPW // K
    NBUF = 3
    num_pairs = bs // 2
    WPI = C // RPW
    WPI_SHIFT = WPI.bit_length() - 1
    assert WPI == 1 << WPI_SHIFT and C % RPW == 0
    mesh = plsc.VectorSubcoreMesh(core_axis_name="c", subcore_axis_name="s")

    @functools.partial(
        pl.kernel,
        mesh=mesh,
        compiler_params=pltpu.CompilerParams(needs_layout_passes=False),
        out_type=jax.ShapeDtypeStruct((bs, C, H, W), jnp.float32),
        scratch_types=[
            pltpu.VMEM((num_pairs * C,), jnp.int32),
            pltpu.VMEM((C,), jnp.int32),
            pltpu.VMEM((NBUF * K, H, W), jnp.float32),
            pltpu.SemaphoreType.DMA((NBUF,)),
            pltpu.SemaphoreType.DMA((NBUF,)),
        ],
    )
    def sc_k(y_hbm, mix_hbm, drop_hbm, out_hbm, mix_v, drop_v, buf_v,
             gsem, wsem):
        wid = lax.axis_index("s") * _NC + lax.axis_index("c")
        inst = wid >> WPI_SHIFT
        cbase = (wid & (WPI - 1)) * RPW
        pair = inst >> 1
        pltpu.sync_copy(mix_hbm, mix_v)
        pltpu.sync_copy(drop_hbm, drop_v)
        lanes = lax.iota(jnp.int32, _L)

        def chunk_flags(g):
            cs = cbase + g * K
            c_vec = jnp.minimum(cs + lanes, C - 1)
            mix_vec = plsc.load_gather(mix_v, [pair * C + c_vec])
            drop_vec = plsc.load_gather(drop_v, [c_vec])
            return mix_vec, drop_vec

        def lane_scalar(vec, j):
            return jnp.sum(jnp.where(lanes == j, vec, 0))

        def start_gather(g):
            b = g % NBUF
            cs = cbase + g * K
            mix_vec, _ = chunk_flags(g)
            for j in range(K):
                src_inst = jnp.where(lane_scalar(mix_vec, j) != 0,
                                     inst ^ 1, inst)
                pltpu.async_copy(
                    y_hbm.at[src_inst, cs + j],
                    buf_v.at[b * K + j], gsem.at[b])

        def wait_gather(g):
            b = g % NBUF
            for j in range(K):
                pltpu.make_async_copy(
                    y_hbm.at[0, 0],
                    buf_v.at[b * K + j], gsem.at[b]).wait()

        def start_write(g):
            b = g % NBUF
            pltpu.async_copy(
                buf_v.at[pl.ds(b * K, K)],
                out_hbm.at[inst, pl.ds(cbase + g * K, K)], wsem.at[b])

        def wait_write(g):
            b = g % NBUF
            pltpu.make_async_copy(
                buf_v.at[pl.ds(b * K, K)],
                out_hbm.at[inst, pl.ds(cbase + g * K, K)],
                wsem.at[b]).wait()

        for g in range(min(NBUF, NCH)):
            start_gather(g)
        for g in range(NCH):
            b = g % NBUF
            if g >= 1 and g - 1 + NBUF < NCH:
                wait_write(g - 1)
                start_gather(g - 1 + NBUF)
            wait_gather(g)
            _, drop_vec = chunk_flags(g)
            for j in range(K):
                is_drop = lane_scalar(drop_vec, j)

                @pl.when(is_drop != 0)
                def _zero_plane(b=b, j=j):
                    zv = jnp.zeros((_L,), jnp.float32)

                    def zbody(q, carry):
                        for l in range(W // _L):
                            buf_v[b * K + j, q, pl.ds(l * _L, _L)] = zv
                        return carry

                    lax.fori_loop(0, H, zbody, 0)

            start_write(g)
        for g in range(max(NCH - NBUF, 0), NCH):
            wait_write(g)

    return sc_k


def kernel(y, mix_mask, drop_mask):
    bs, C, H, W = y.shape
    num_pairs = bs // 2
    RPW = bs * C // _NW
    K = 4
    mix_i = mix_mask.astype(jnp.int32).reshape(num_pairs * C)
    drop_i = drop_mask.astype(jnp.int32)
    sc_k = _build_sc_kernel(bs, C, H, W, RPW, K)
    return sc_k(y, mix_i, drop_i)
